# baseline (speedup 1.0000x reference)
import functools

import jax
import jax.numpy as jnp
from jax import lax
from jax.experimental import pallas as pl
from jax.experimental.pallas import tpu as pltpu
from jax.experimental.pallas import tpu_sc as plsc

DIM = 256
DW = DIM // 2
HID = 512
N = 10000
K = 16
EPS = 1e-5

NC, NS, L = 2, 16, 16
NPAD = 10240
CH = 8
NCHT = NPAD // CH
CA = 80
CB = 80 - CA

RB = 1024
GRID = NPAD // RB

_BN_S = (1.0 + EPS) ** -0.5


def _run_stage(fn, acts, consts, outs):
    def spec(shape):
        return pl.BlockSpec((RB, shape[1]), lambda i: (i, 0))

    out_shape = tuple(jax.ShapeDtypeStruct(s, dt) for s, dt in outs)
    specs_in = ([spec(a.shape) for a in acts] +
                [pl.BlockSpec(c.shape, lambda i, _r=len(c.shape): (0,) * _r)
                 for c in consts])
    return pl.pallas_call(
        fn,
        grid=(GRID,),
        in_specs=specs_in,
        out_specs=tuple(spec(s) for s, _ in outs),
        out_shape=out_shape,
    )(*acts, *consts)


def _gelu_exact(x):
    return 0.5 * x * (1.0 + lax.erf(x * jnp.float32(0.7071067811865476)))


def _pack_px(y):
    yb = y.astype(jnp.bfloat16)
    lo = lax.bitcast_convert_type(yb[:, :DW], jnp.uint16).astype(jnp.uint32)
    hi = lax.bitcast_convert_type(yb[:, DW:], jnp.uint16).astype(jnp.uint32)
    return lax.bitcast_convert_type(lo | (hi << 16), jnp.int32)


def _unpack_xk(w):
    u = lax.bitcast_convert_type(w, jnp.uint32)
    lo = lax.bitcast_convert_type((u & jnp.uint32(0xFFFF)).astype(jnp.uint16),
                                  jnp.bfloat16)
    hi = lax.bitcast_convert_type((u >> 16).astype(jnp.uint16), jnp.bfloat16)
    return jnp.concatenate([lo, hi], axis=1)


def _bdot(a, w):
    return jnp.dot(a.astype(jnp.bfloat16), w,
                   preferred_element_type=jnp.float32)


def _mlp_f(x, w1t, b1, w2t, g, b):
    h = _gelu_exact(_bdot(x, w1t) + b1)
    return _bdot(h, w2t) * g + b


def _dfil_rest_f(x, xk, lwt, lg, lb, gwt, gg, gb, a1wt, a1g, a1c, a2wt, a2g,
                 a2c, bg, bb):
    x1 = _bdot(xk, lwt) * lg + lb
    x2 = _bdot(x, gwt) * gg + gb
    f = x1 + x2
    t = _bdot(f, a1wt) * a1g + a1c
    t = _bdot(t, a2wt) * a2g + a2c
    t = jax.nn.sigmoid(t)
    out = t * x1 + (1.0 - t) * x2
    return out * bg + bb


def _stage_first(x_ref, w1t, b1, w2t, g, b, pwt, xo_ref, px_ref):
    x = x_ref[...]
    xn = x + _mlp_f(x, w1t[...], b1[...], w2t[...], g[...], b[...])
    xo_ref[...] = xn
    px_ref[...] = _pack_px(_bdot(xn, pwt[...]))


def _stage_mid(x_ref, xk_ref, lwt, lg, lb, gwt, gg, gb, a1wt, a1g, a1c, a2wt,
               a2g, a2c, bg, bb, pwt, xo_ref, px_ref):
    x = x_ref[...]
    xn = x + _dfil_rest_f(x, _unpack_xk(xk_ref[...]), lwt[...], lg[...],
                          lb[...], gwt[...], gg[...], gb[...], a1wt[...],
                          a1g[...], a1c[...], a2wt[...], a2g[...], a2c[...],
                          bg[...], bb[...])
    xo_ref[...] = xn
    px_ref[...] = _pack_px(_bdot(xn, pwt[...]))


def _stage_mid_mlp(x_ref, xk_ref, lwt, lg, lb, gwt, gg, gb, a1wt, a1g, a1c,
                   a2wt, a2g, a2c, bg, bb, w1t, b1, w2t, mg, mb, pwt, xo_ref,
                   px_ref):
    x = x_ref[...]
    xn = x + _dfil_rest_f(x, _unpack_xk(xk_ref[...]), lwt[...], lg[...],
                          lb[...], gwt[...], gg[...], gb[...], a1wt[...],
                          a1g[...], a1c[...], a2wt[...], a2g[...], a2c[...],
                          bg[...], bb[...])
    xn = xn + _mlp_f(xn, w1t[...], b1[...], w2t[...], mg[...], mb[...])
    xo_ref[...] = xn
    px_ref[...] = _pack_px(_bdot(xn, pwt[...]))


def _stage_last(x_ref, xk_ref, lwt, lg, lb, gwt, gg, gb, a1wt, a1g, a1c, a2wt,
                a2g, a2c, bg, bb, w1t, b1, w2t, mg, mb, xo_ref):
    x = x_ref[...]
    xn = x + _dfil_rest_f(x, _unpack_xk(xk_ref[...]), lwt[...], lg[...],
                          lb[...], gwt[...], gg[...], gb[...], a1wt[...],
                          a1g[...], a1c[...], a2wt[...], a2g[...], a2c[...],
                          bg[...], bb[...])
    xn = xn + _mlp_f(xn, w1t[...], b1[...], w2t[...], mg[...], mb[...])
    xo_ref[...] = xn


@functools.cache
def _make_gather_max_sc():
    cmax = max(CA, CB)

    @functools.partial(
        pl.kernel,
        out_type=jax.ShapeDtypeStruct((NPAD, DW), jnp.int32),
        mesh=plsc.VectorSubcoreMesh(core_axis_name="c", subcore_axis_name="s",
                                    num_cores=1, num_subcores=NS),
        scratch_types=[
            pltpu.VMEM((cmax, CH * K), jnp.int32),
            pltpu.VMEM((CH * K, DW), jnp.int32),
            pltpu.VMEM((CH * K, DW), jnp.int32),
            pltpu.VMEM((CH * K, DW), jnp.int32),
            pltpu.VMEM((CH * K, DW), jnp.int32),
            pltpu.VMEM((CH * K, DW), jnp.int32),
            pltpu.VMEM((CH, DW), jnp.int32),
            pltpu.VMEM((CH, DW), jnp.int32),
            pltpu.SemaphoreType.DMA,
            pltpu.SemaphoreType.DMA,
            pltpu.SemaphoreType.DMA,
            pltpu.SemaphoreType.DMA,
            pltpu.SemaphoreType.DMA,
            pltpu.SemaphoreType.DMA,
            pltpu.SemaphoreType.DMA,
        ],
        compiler_params=pltpu.CompilerParams(needs_layout_passes=False),
    )
    def gather_max(px_hbm, idx_hbm, out_hbm, idx_all, rows0, rows1, rows2,
                   rows3, rows4, ob0, ob1, sem0, sem1, sem2, sem3, sem4,
                   semo0, semo1):
        sub = lax.axis_index("s")
        rows = (rows0, rows1, rows2, rows3, rows4)
        sems = (sem0, sem1, sem2, sem3, sem4)
        obufs = (ob0, ob1)
        osems = (semo0, semo1)

        def compute_chunk(rbuf, obuf):
            def node_body(n, carry):
                r0 = n * K
                for q in range(DW // L):
                    sl = pl.ds(q * L, L)
                    acc = plsc.bitcast(rbuf[r0, sl], jnp.bfloat16)
                    for j in range(1, K):
                        acc = jnp.maximum(
                            acc, plsc.bitcast(rbuf[r0 + j, sl], jnp.bfloat16))
                    obuf[n, sl] = plsc.bitcast(acc, jnp.int32)
                return carry

            lax.fori_loop(0, CH, node_body, 0)

        def worker(cbase, nch):
            pltpu.sync_copy(idx_hbm.at[pl.ds(cbase, nch)],
                            idx_all.at[pl.ds(0, nch)])
            for j in range(4):
                pltpu.async_copy(px_hbm.at[idx_all.at[j]], rows[j], sems[j])

            def out_slice(c):
                return out_hbm.at[pl.ds((cbase + c) * CH, CH)]

            def loop_body(i, carry):
                c = i * 5
                for j in range(5):
                    cc = c + j
                    rb, sm = rows[j], sems[j]
                    ob, so = obufs[j % 2], osems[j % 2]
                    pltpu.make_async_copy(px_hbm.at[idx_all.at[cc]], rb,
                                          sm).wait()

                    @pl.when(cc >= 2)
                    def _drain_out():
                        pltpu.make_async_copy(ob, out_slice(cc - 2),
                                              so).wait()

                    compute_chunk(rb, ob)
                    pltpu.async_copy(ob, out_slice(cc), so)

                    nxt = cc + 4
                    nb = (j + 4) % 5

                    @pl.when(nxt < nch)
                    def _issue_next():
                        pltpu.async_copy(px_hbm.at[idx_all.at[nxt]], rows[nb],
                                         sems[nb])

                return carry

            lax.fori_loop(0, nch // 5, loop_body, 0)
            pltpu.make_async_copy(ob0, out_slice(nch - 2), semo0).wait()
            pltpu.make_async_copy(ob1, out_slice(nch - 1), semo1).wait()

        worker(sub * CA, CA)

    return gather_max


def _gather_max_sc(px_i, idx_chunks):
    return _make_gather_max_sc()(px_i, idx_chunks)


def _prep(params):
    s = jnp.float32(_BN_S)

    def mlp_c(p):
        return (p['W1'].T.astype(jnp.bfloat16), p['b1'][None, :],
                p['W2'].T.astype(jnp.bfloat16),
                (s * p['bn_g'])[None, :], p['bn_b'][None, :])

    def dfil_c(p):
        a1g = s * p['aff_g1']
        a2g = s * p['aff_g2']
        bt = lambda w: w.T.astype(jnp.bfloat16)
        return (bt(p['local_W']), (s * p['local_g'])[None, :], p['local_b'][None, :],
                bt(p['glob_W']), (s * p['glob_g'])[None, :], p['glob_b'][None, :],
                bt(p['aff_W1']), a1g[None, :], (p['aff_b1'] * a1g + p['aff_bb1'])[None, :],
                bt(p['aff_W2']), a2g[None, :], (p['aff_b2'] * a2g + p['aff_bb2'])[None, :],
                (s * p['bn_g'])[None, :], p['bn_b'][None, :])

    return {
        'mlp0': mlp_c(params['mlp0']),
        'mlps': [mlp_c(p) for p in params['mlps']],
        'dfils': [dfil_c(p) for p in params['dfils']],
        'projs': [p['proj_W'].T.astype(jnp.bfloat16) for p in params['dfils']],
    }


def kernel(x, params, knn):
    c = _prep(params)
    x0 = x[0]
    pad_idx = (jnp.arange((NPAD - N) * K, dtype=jnp.int32) * 41) % N
    idx = jnp.concatenate(
        [knn[0].astype(jnp.int32).reshape(-1), pad_idx]).reshape(NCHT, CH * K)

    xs_f = ((NPAD, DIM), jnp.float32)
    px_i = ((NPAD, DW), jnp.int32)
    xc, px = _run_stage(_stage_first, [x0], list(c['mlp0']) + [c['projs'][0]],
                        (xs_f, px_i))

    xk = _gather_max_sc(px, idx)
    xc, px = _run_stage(_stage_mid, [xc, xk],
                        list(c['dfils'][0]) + [c['projs'][1]], (xs_f, px_i))

    xk = _gather_max_sc(px, idx)
    xc, px = _run_stage(_stage_mid_mlp, [xc, xk],
                        list(c['dfils'][1]) + list(c['mlps'][0]) + [c['projs'][2]],
                        (xs_f, px_i))

    xk = _gather_max_sc(px, idx)
    xc, px = _run_stage(_stage_mid, [xc, xk],
                        list(c['dfils'][2]) + [c['projs'][3]], (xs_f, px_i))

    xk = _gather_max_sc(px, idx)
    (out,) = _run_stage(_stage_last, [xc, xk],
                        list(c['dfils'][3]) + list(c['mlps'][1]),
                        (((N, DIM), jnp.float32),))

    return out[None]


# speedup vs baseline: 1.0048x; 1.0048x over previous
import functools

import jax
import jax.numpy as jnp
from jax import lax
from jax.experimental import pallas as pl
from jax.experimental.pallas import tpu as pltpu
from jax.experimental.pallas import tpu_sc as plsc

DIM = 256
DW = DIM // 2
HID = 512
N = 10000
K = 16
EPS = 1e-5

NC, NS, L = 2, 16, 16
NPAD = 10240
CH = 8
NCHT = NPAD // CH
CA = NCHT // NS

RB = 1024
GRID = NPAD // RB

_BN_S = (1.0 + EPS) ** -0.5


def _run_stage(fn, acts, consts, outs):
    def spec(shape):
        return pl.BlockSpec((RB, shape[1]), lambda i: (i, 0))

    out_shape = tuple(jax.ShapeDtypeStruct(s, dt) for s, dt in outs)
    specs_in = ([spec(a.shape) for a in acts] +
                [pl.BlockSpec(c.shape, lambda i, _r=len(c.shape): (0,) * _r)
                 for c in consts])
    return pl.pallas_call(
        fn,
        grid=(GRID,),
        in_specs=specs_in,
        out_specs=tuple(spec(s) for s, _ in outs),
        out_shape=out_shape,
    )(*acts, *consts)


def _gelu_exact(x):
    return 0.5 * x * (1.0 + lax.erf(x * jnp.float32(0.7071067811865476)))


def _pack_px(y):
    yb = y.astype(jnp.bfloat16)
    lo = lax.bitcast_convert_type(yb[:, :DW], jnp.uint16).astype(jnp.uint32)
    hi = lax.bitcast_convert_type(yb[:, DW:], jnp.uint16).astype(jnp.uint32)
    return lax.bitcast_convert_type(lo | (hi << 16), jnp.int32)


def _unpack_xk(w):
    u = lax.bitcast_convert_type(w, jnp.uint32)
    lo = lax.bitcast_convert_type((u & jnp.uint32(0xFFFF)).astype(jnp.uint16),
                                  jnp.bfloat16)
    hi = lax.bitcast_convert_type((u >> 16).astype(jnp.uint16), jnp.bfloat16)
    return jnp.concatenate([lo, hi], axis=1)


def _bdot(a, w):
    return jnp.dot(a.astype(jnp.bfloat16), w,
                   preferred_element_type=jnp.float32)


def _mlp_f(x, w1t, b1, w2t, g, b):
    h = _gelu_exact(_bdot(x, w1t) + b1)
    return _bdot(h, w2t) * g + b


def _dfil_rest_f(x, xk, lwt, lg, lb, gwt, gg, gb, a1wt, a1g, a1c, a2wt, a2g,
                 a2c, bg, bb):
    x1 = _bdot(xk, lwt) * lg + lb
    x2 = _bdot(x, gwt) * gg + gb
    f = x1 + x2
    t = _bdot(f, a1wt) * a1g + a1c
    t = _bdot(t, a2wt) * a2g + a2c
    t = jax.nn.sigmoid(t)
    out = t * x1 + (1.0 - t) * x2
    return out * bg + bb


def _stage_first(x_ref, w1t, b1, w2t, g, b, pwt, xo_ref, px_ref):
    x = x_ref[...]
    xn = x + _mlp_f(x, w1t[...], b1[...], w2t[...], g[...], b[...])
    xo_ref[...] = xn
    px_ref[...] = _pack_px(_bdot(xn, pwt[...]))


def _stage_mid(x_ref, xk_ref, lwt, lg, lb, gwt, gg, gb, a1wt, a1g, a1c, a2wt,
               a2g, a2c, bg, bb, pwt, xo_ref, px_ref):
    x = x_ref[...]
    xn = x + _dfil_rest_f(x, _unpack_xk(xk_ref[...]), lwt[...], lg[...],
                          lb[...], gwt[...], gg[...], gb[...], a1wt[...],
                          a1g[...], a1c[...], a2wt[...], a2g[...], a2c[...],
                          bg[...], bb[...])
    xo_ref[...] = xn
    px_ref[...] = _pack_px(_bdot(xn, pwt[...]))


def _stage_mid_mlp(x_ref, xk_ref, lwt, lg, lb, gwt, gg, gb, a1wt, a1g, a1c,
                   a2wt, a2g, a2c, bg, bb, w1t, b1, w2t, mg, mb, pwt, xo_ref,
                   px_ref):
    x = x_ref[...]
    xn = x + _dfil_rest_f(x, _unpack_xk(xk_ref[...]), lwt[...], lg[...],
                          lb[...], gwt[...], gg[...], gb[...], a1wt[...],
                          a1g[...], a1c[...], a2wt[...], a2g[...], a2c[...],
                          bg[...], bb[...])
    xn = xn + _mlp_f(xn, w1t[...], b1[...], w2t[...], mg[...], mb[...])
    xo_ref[...] = xn
    px_ref[...] = _pack_px(_bdot(xn, pwt[...]))


def _stage_last(x_ref, xk_ref, lwt, lg, lb, gwt, gg, gb, a1wt, a1g, a1c, a2wt,
                a2g, a2c, bg, bb, w1t, b1, w2t, mg, mb, xo_ref):
    x = x_ref[...]
    xn = x + _dfil_rest_f(x, _unpack_xk(xk_ref[...]), lwt[...], lg[...],
                          lb[...], gwt[...], gg[...], gb[...], a1wt[...],
                          a1g[...], a1c[...], a2wt[...], a2g[...], a2c[...],
                          bg[...], bb[...])
    xn = xn + _mlp_f(xn, w1t[...], b1[...], w2t[...], mg[...], mb[...])
    xo_ref[...] = xn


@functools.cache
def _make_gather_max_sc():
    cmax = CA

    @functools.partial(
        pl.kernel,
        out_type=jax.ShapeDtypeStruct((NPAD, DW), jnp.int32),
        mesh=plsc.VectorSubcoreMesh(core_axis_name="c", subcore_axis_name="s",
                                    num_cores=1, num_subcores=NS),
        scratch_types=[
            pltpu.VMEM((cmax, CH * K), jnp.int32),
            pltpu.VMEM((CH * K, DW), jnp.int32),
            pltpu.VMEM((CH * K, DW), jnp.int32),
            pltpu.VMEM((CH * K, DW), jnp.int32),
            pltpu.VMEM((CH * K, DW), jnp.int32),
            pltpu.VMEM((CH * K, DW), jnp.int32),
            pltpu.VMEM((CH, DW), jnp.int32),
            pltpu.VMEM((CH, DW), jnp.int32),
            pltpu.SemaphoreType.DMA,
            pltpu.SemaphoreType.DMA,
            pltpu.SemaphoreType.DMA,
            pltpu.SemaphoreType.DMA,
            pltpu.SemaphoreType.DMA,
            pltpu.SemaphoreType.DMA,
            pltpu.SemaphoreType.DMA,
        ],
        compiler_params=pltpu.CompilerParams(needs_layout_passes=False),
    )
    def gather_max(px_hbm, idx_hbm, out_hbm, idx_all, rows0, rows1, rows2,
                   rows3, rows4, ob0, ob1, sem0, sem1, sem2, sem3, sem4,
                   semo0, semo1):
        sub = lax.axis_index("s")
        rows = (rows0, rows1, rows2, rows3, rows4)
        sems = (sem0, sem1, sem2, sem3, sem4)
        obufs = (ob0, ob1)
        osems = (semo0, semo1)

        def compute_chunk(rbuf, obuf):
            def node_body(n, carry):
                r0 = n * K
                for q in range(DW // L):
                    sl = pl.ds(q * L, L)
                    acc = plsc.bitcast(rbuf[r0, sl], jnp.bfloat16)
                    for j in range(1, K):
                        acc = jnp.maximum(
                            acc, plsc.bitcast(rbuf[r0 + j, sl], jnp.bfloat16))
                    obuf[n, sl] = plsc.bitcast(acc, jnp.int32)
                return carry

            lax.fori_loop(0, CH, node_body, 0)

        def worker(cbase, nch):
            pltpu.sync_copy(idx_hbm.at[pl.ds(cbase, nch)],
                            idx_all.at[pl.ds(0, nch)])
            for j in range(4):
                pltpu.async_copy(px_hbm.at[idx_all.at[j]], rows[j], sems[j])

            def out_slice(c):
                return out_hbm.at[pl.ds((cbase + c) * CH, CH)]

            def loop_body(i, carry):
                c = i * 5
                for j in range(5):
                    cc = c + j
                    rb, sm = rows[j], sems[j]
                    ob, so = obufs[j % 2], osems[j % 2]
                    pltpu.make_async_copy(px_hbm.at[idx_all.at[cc]], rb,
                                          sm).wait()

                    @pl.when(cc >= 2)
                    def _drain_out():
                        pltpu.make_async_copy(ob, out_slice(cc - 2),
                                              so).wait()

                    compute_chunk(rb, ob)
                    pltpu.async_copy(ob, out_slice(cc), so)

                    nxt = cc + 4
                    nb = (j + 4) % 5

                    @pl.when(nxt < nch)
                    def _issue_next():
                        pltpu.async_copy(px_hbm.at[idx_all.at[nxt]], rows[nb],
                                         sems[nb])

                return carry

            lax.fori_loop(0, nch // 5, loop_body, 0)
            pltpu.make_async_copy(ob0, out_slice(nch - 2), semo0).wait()
            pltpu.make_async_copy(ob1, out_slice(nch - 1), semo1).wait()

        worker(sub * CA, CA)

    return gather_max


def _gather_max_sc(px_i, idx_chunks):
    return _make_gather_max_sc()(px_i, idx_chunks)


def _prep(params):
    s = jnp.float32(_BN_S)

    def mlp_c(p):
        return (p['W1'].T.astype(jnp.bfloat16), p['b1'][None, :],
                p['W2'].T.astype(jnp.bfloat16),
                (s * p['bn_g'])[None, :], p['bn_b'][None, :])

    def dfil_c(p):
        a1g = s * p['aff_g1']
        a2g = s * p['aff_g2']
        bt = lambda w: w.T.astype(jnp.bfloat16)
        return (bt(p['local_W']), (s * p['local_g'])[None, :], p['local_b'][None, :],
                bt(p['glob_W']), (s * p['glob_g'])[None, :], p['glob_b'][None, :],
                bt(p['aff_W1']), a1g[None, :], (p['aff_b1'] * a1g + p['aff_bb1'])[None, :],
                bt(p['aff_W2']), a2g[None, :], (p['aff_b2'] * a2g + p['aff_bb2'])[None, :],
                (s * p['bn_g'])[None, :], p['bn_b'][None, :])

    return {
        'mlp0': mlp_c(params['mlp0']),
        'mlps': [mlp_c(p) for p in params['mlps']],
        'dfils': [dfil_c(p) for p in params['dfils']],
        'projs': [p['proj_W'].T.astype(jnp.bfloat16) for p in params['dfils']],
    }


def kernel(x, params, knn):
    c = _prep(params)
    x0 = x[0]
    pad_idx = (jnp.arange((NPAD - N) * K, dtype=jnp.int32) * 41) % N
    idx = jnp.concatenate(
        [knn[0].astype(jnp.int32).reshape(-1), pad_idx]).reshape(NCHT, CH * K)

    xs_f = ((NPAD, DIM), jnp.float32)
    px_i = ((NPAD, DW), jnp.int32)
    xc, px = _run_stage(_stage_first, [x0], list(c['mlp0']) + [c['projs'][0]],
                        (xs_f, px_i))

    xk = _gather_max_sc(px, idx)
    xc, px = _run_stage(_stage_mid, [xc, xk],
                        list(c['dfils'][0]) + [c['projs'][1]], (xs_f, px_i))

    xk = _gather_max_sc(px, idx)
    xc, px = _run_stage(_stage_mid_mlp, [xc, xk],
                        list(c['dfils'][1]) + list(c['mlps'][0]) + [c['projs'][2]],
                        (xs_f, px_i))

    xk = _gather_max_sc(px, idx)
    xc, px = _run_stage(_stage_mid, [xc, xk],
                        list(c['dfils'][2]) + [c['projs'][3]], (xs_f, px_i))

    xk = _gather_max_sc(px, idx)
    (out,) = _run_stage(_stage_last, [xc, xk],
                        list(c['dfils'][3]) + list(c['mlps'][1]),
                        (((N, DIM), jnp.float32),))

    return out[None]
